# Initial kernel scaffold; baseline (speedup 1.0000x reference)
import jax
import jax.numpy as jnp
from jax.experimental import pallas as pl


def kernel(input_ids, attention_mask, domain_ids, token_table, prefix_table):
    raise NotImplementedError("write your pallas kernel here")


# speedup vs baseline: 1.9920x; 1.9920x over previous
import functools

import jax
import jax.numpy as jnp
from jax import lax
from jax.experimental import pallas as pl
from jax.experimental.pallas import tpu as pltpu
from jax.experimental.pallas import tpu_sc as plsc

_NUM_DOMAINS = 16
_PREFIX_LEN = 32
_HIDDEN = 2048
_VOCAB = 32000
_BATCH = 4
_SEQ = 2048

_NC, _NS = 2, 16
_NW = _NC * _NS
_TOK = _BATCH * _SEQ
_TPW = _TOK // _NW
_WPB = _NW // _BATCH
_CHUNK = 16
_NCHUNK = _TPW // _CHUNK
_NBUF = 3
_PPW = 8
_PREF_WORKERS = _BATCH * _PREFIX_LEN // _PPW


def _body(ids_hbm, pidx_hbm, mask_hbm, tok_hbm, pref_hbm,
          out_e_hbm, out_m_hbm,
          idx_v, bufs_v, pidx_v, prow_v, ones_v, mask_v,
          g0, g1, g2, p0, p1, p2):
  gsem = [g0, g1, g2]
  psem = [p0, p1, p2]
  c = lax.axis_index("c")
  s = lax.axis_index("s")
  w = c * _NS + s
  b = w // _WPB
  s0 = (w % _WPB) * _TPW

  pltpu.sync_copy(ids_hbm.at[pl.ds(w * _TPW, _TPW)], idx_v)

  m0 = b * (_PREFIX_LEN + _SEQ)
  pltpu.sync_copy(mask_hbm.at[pl.ds(w * _TPW, _TPW)], mask_v)
  pltpu.sync_copy(mask_v, out_m_hbm.at[pl.ds(m0 + _PREFIX_LEN + s0, _TPW)])

  ones_v[pl.ds(0, 16)] = jnp.ones((16,), jnp.int32)
  ones_v[pl.ds(16, 16)] = jnp.ones((16,), jnp.int32)

  @pl.when(w < _BATCH)
  def _():
    pltpu.sync_copy(ones_v,
                    out_m_hbm.at[pl.ds(w * (_PREFIX_LEN + _SEQ), _PREFIX_LEN)])

  @pl.when(w < _PREF_WORKERS)
  def _():
    pltpu.sync_copy(pidx_hbm.at[pl.ds(w * _PPW, _PPW)], pidx_v)
    pltpu.async_copy(pref_hbm.at[pidx_v], prow_v, g0).wait()
    b2 = w // (_PREFIX_LEN // _PPW)
    pp0 = (w % (_PREFIX_LEN // _PPW)) * _PPW
    pltpu.sync_copy(prow_v, out_e_hbm.at[b2, pl.ds(pp0, _PPW)])

  gdesc = [None] * _NCHUNK
  pdesc = [None] * _NCHUNK

  def start_gather(j):
    k = j % _NBUF
    gdesc[j] = pltpu.async_copy(
        tok_hbm.at[idx_v.at[pl.ds(j * _CHUNK, _CHUNK)]], bufs_v.at[k],
        gsem[k])

  for j in range(min(_NBUF, _NCHUNK)):
    start_gather(j)

  for j in range(_NCHUNK):
    k = j % _NBUF
    if 1 <= j and j + 2 < _NCHUNK:
      pdesc[j - 1].wait()
      start_gather(j + 2)
    gdesc[j].wait()
    r0 = _PREFIX_LEN + s0 + j * _CHUNK
    pdesc[j] = pltpu.async_copy(
        bufs_v.at[k], out_e_hbm.at[b, pl.ds(r0, _CHUNK)], psem[k])

  for j in range(max(0, _NCHUNK - 3), _NCHUNK):
    pdesc[j].wait()


@jax.jit
def _sc_embed(ids, pidx, mask, token_table, pref2d):
  mesh = plsc.VectorSubcoreMesh(core_axis_name="c", subcore_axis_name="s")
  fn = functools.partial(
      pl.kernel,
      out_type=(
          jax.ShapeDtypeStruct((_BATCH, _PREFIX_LEN + _SEQ, _HIDDEN),
                               jnp.float32),
          jax.ShapeDtypeStruct((_BATCH * (_PREFIX_LEN + _SEQ),), jnp.int32),
      ),
      mesh=mesh,
      scratch_types=[
          pltpu.VMEM((_TPW,), jnp.int32),
          pltpu.VMEM((_NBUF, _CHUNK, _HIDDEN), jnp.float32),
          pltpu.VMEM((_PPW,), jnp.int32),
          pltpu.VMEM((_PPW, _HIDDEN), jnp.float32),
          pltpu.VMEM((_PREFIX_LEN,), jnp.int32),
          pltpu.VMEM((_TPW,), jnp.int32),
          pltpu.SemaphoreType.DMA,
          pltpu.SemaphoreType.DMA,
          pltpu.SemaphoreType.DMA,
          pltpu.SemaphoreType.DMA,
          pltpu.SemaphoreType.DMA,
          pltpu.SemaphoreType.DMA,
      ],
  )(_body)
  return fn(ids, pidx, mask, token_table, pref2d)


def kernel(input_ids, attention_mask, domain_ids, token_table, prefix_table):
  mask_dtype = attention_mask.dtype
  ids = input_ids.astype(jnp.int32).reshape(_TOK)
  mask = attention_mask.astype(jnp.int32).reshape(_TOK)
  dom = domain_ids.astype(jnp.int32)
  pidx = (dom[:, None] * _PREFIX_LEN
          + jnp.arange(_PREFIX_LEN, dtype=jnp.int32)[None, :]).reshape(-1)
  pref2d = prefix_table.reshape(_NUM_DOMAINS * _PREFIX_LEN, _HIDDEN)
  out_e, out_m = _sc_embed(ids, pidx, mask, token_table, pref2d)
  out_m = out_m.reshape(_BATCH, _PREFIX_LEN + _SEQ).astype(mask_dtype)
  return out_e, out_m
